# baseline (speedup 1.0000x reference)
import functools

import jax
import jax.numpy as jnp
from jax import lax
from jax.experimental import pallas as pl
from jax.experimental.pallas import tpu as pltpu
from jax.experimental.pallas import tpu_sc as plsc

_W = 2
_K = 5
_KK = _K * _K


def kernel(grid_input, w, memory):
    B = grid_input.shape[0]
    N, M, H = memory.shape
    assert B == 4096 and N == 512 and M == 512 and H == 256
    table = memory.reshape(N * M, H)
    grid_adj = grid_input + (w - _W)
    qx = grid_adj[:, 0]
    qy = grid_adj[:, 1]
    def permute_coords(q):
        qt = jnp.tile(q, _KK)[: B * _KK].reshape(B, _KK).T
        row25 = jnp.roll(qt[0], -1)[None, :]
        return jnp.concatenate([qt, row25], axis=0)

    qxp = permute_coords(qx)
    qyp = permute_coords(qy)

    R = B * _KK
    NC, NS, L = 2, 16, 16
    NW = NC * NS
    rows_per_w = R // NW
    CHUNK = 128
    n_chunks = rows_per_w // CHUNK

    mesh = plsc.VectorSubcoreMesh(core_axis_name="c", subcore_axis_name="s")

    @functools.partial(
        pl.kernel,
        mesh=mesh,
        compiler_params=pltpu.CompilerParams(use_tc_tiling_on_sc=True),
        out_type=jax.ShapeDtypeStruct((R, H), jnp.float32),
        scratch_types=[
            pltpu.VMEM((2 * B,), jnp.int32),
            pltpu.VMEM((2 * B,), jnp.int32),
            pltpu.VMEM((CHUNK,), jnp.int32),
            pltpu.VMEM((CHUNK,), jnp.int32),
            pltpu.VMEM((CHUNK,), jnp.int32),
            pltpu.VMEM((CHUNK, H), jnp.float32),
            pltpu.VMEM((CHUNK, H), jnp.float32),
            pltpu.VMEM((CHUNK, H), jnp.float32),
            pltpu.SemaphoreType.DMA,
            pltpu.SemaphoreType.DMA,
            pltpu.SemaphoreType.DMA,
            pltpu.SemaphoreType.DMA,
            pltpu.SemaphoreType.DMA,
            pltpu.SemaphoreType.DMA,
        ],
    )
    def gather_rows(qxp_hbm, qyp_hbm, table_hbm, out_hbm,
                    cx_v, cy_v, idx0_v, idx1_v, idx2_v,
                    rows0_v, rows1_v, rows2_v,
                    sg0, sg1, sg2, sw0, sw1, sw2):
        wid = lax.axis_index("s") * NC + lax.axis_index("c")
        base = wid * rows_per_w
        p_lo = lax.shift_right_logical(base, 12)
        pltpu.sync_copy(qxp_hbm.at[p_lo], cx_v.at[pl.ds(0, B)])
        pltpu.sync_copy(qxp_hbm.at[p_lo + 1], cx_v.at[pl.ds(B, B)])
        pltpu.sync_copy(qyp_hbm.at[p_lo], cy_v.at[pl.ds(0, B)])
        pltpu.sync_copy(qyp_hbm.at[p_lo + 1], cy_v.at[pl.ds(B, B)])
        lanes = lax.iota(jnp.int32, L)

        idx_v = (idx0_v, idx1_v, idx2_v)
        rows_v = (rows0_v, rows1_v, rows2_v)
        sg = (sg0, sg1, sg2)
        sw = (sw0, sw1, sw2)

        def compute_idx(c, dst):
            start = base + c * CHUNK
            p = lax.shift_right_logical(start, 12)
            b0 = jnp.bitwise_and(start, B - 1)
            off = lax.shift_left(p - p_lo, 12) + b0
            for v in range(CHUNK // L):
                bv = b0 + v * L + lanes
                f = bv * _KK + p
                q = lax.shift_right_logical(f, 12)
                i = lax.shift_right_logical(q * 13108, 16)
                j = q - i * _K
                gx = cx_v[pl.ds(off + v * L, L)]
                gy = cy_v[pl.ds(off + v * L, L)]
                rx = jnp.bitwise_and(gx + (i - _W), N - 1)
                ry = jnp.bitwise_and(gy + (j - _W), M - 1)
                dst[pl.ds(v * L, L)] = lax.shift_left(rx, 9) + ry

        def fire_gather(p):
            pltpu.async_copy(table_hbm.at[idx_v[p]], rows_v[p], sg[p])

        def wait_gather(p):
            pltpu.make_async_copy(table_hbm.at[idx_v[p]], rows_v[p],
                                  sg[p]).wait()

        def fire_wb(c, p):
            pltpu.async_copy(rows_v[p],
                             out_hbm.at[pl.ds(base + c * CHUNK, CHUNK)], sw[p])

        def wait_wb(c, p):
            pltpu.make_async_copy(rows_v[p],
                                  out_hbm.at[pl.ds(base + c * CHUNK, CHUNK)],
                                  sw[p]).wait()

        def step(c, p):
            wait_wb(c - 1, (p + 2) % 3)
            fire_gather((p + 2) % 3)
            wait_gather(p)
            fire_wb(c, p)
            compute_idx(c + 3, idx_v[p])

        compute_idx(0, idx_v[0])
        fire_gather(0)
        compute_idx(1, idx_v[1])
        fire_gather(1)
        compute_idx(2, idx_v[2])
        fire_gather(2)
        wait_gather(0)
        fire_wb(0, 0)
        compute_idx(3, idx_v[0])
        for c, p in ((1, 1), (2, 2)):
            step(c, p)

        def body(k, carry):
            c0 = k * 3
            step(c0, 0)
            step(c0 + 1, 1)
            step(c0 + 2, 2)
            return carry

        lax.fori_loop(1, 7, body, 0)
        wait_wb(20, 2)
        fire_gather(2)
        wait_gather(0)
        fire_wb(21, 0)
        compute_idx(24, idx_v[0])
        wait_wb(21, 0)
        fire_gather(0)
        wait_gather(1)
        fire_wb(22, 1)
        wait_wb(22, 1)
        wait_gather(2)
        fire_wb(23, 2)
        wait_wb(23, 2)
        wait_gather(0)
        fire_wb(24, 0)
        wait_wb(24, 0)

    out = gather_rows(qxp, qyp, table)
    return jnp.transpose(out.reshape(_KK, B, H), (1, 0, 2))


# speedup vs baseline: 1.0625x; 1.0625x over previous
import functools

import jax
import jax.numpy as jnp
from jax import lax
from jax.experimental import pallas as pl
from jax.experimental.pallas import tpu as pltpu
from jax.experimental.pallas import tpu_sc as plsc

_W = 2
_K = 5
_KK = _K * _K


def kernel(grid_input, w, memory):
    B = grid_input.shape[0]
    N, M, H = memory.shape
    assert B == 4096 and N == 512 and M == 512 and H == 256
    table = memory.reshape(N * M, H)
    grid_adj = grid_input + (w - _W)
    qpk = grid_adj[:, 0] | (grid_adj[:, 1] << 16)
    qt = jnp.tile(qpk, _KK).reshape(B, _KK).T
    qp = jnp.concatenate([qt, jnp.roll(qt[0], -1)[None, :]], axis=0)

    R = B * _KK
    NC, NS, L = 2, 16, 16
    NW = NC * NS
    rows_per_w = R // NW
    CHUNK = 128
    n_chunks = rows_per_w // CHUNK

    mesh = plsc.VectorSubcoreMesh(core_axis_name="c", subcore_axis_name="s")

    @functools.partial(
        pl.kernel,
        mesh=mesh,
        compiler_params=pltpu.CompilerParams(use_tc_tiling_on_sc=True),
        out_type=jax.ShapeDtypeStruct((R, H), jnp.float32),
        scratch_types=[
            pltpu.VMEM((2 * B,), jnp.int32),
            pltpu.VMEM((CHUNK,), jnp.int32),
            pltpu.VMEM((CHUNK,), jnp.int32),
            pltpu.VMEM((CHUNK,), jnp.int32),
            pltpu.VMEM((CHUNK, H), jnp.float32),
            pltpu.VMEM((CHUNK, H), jnp.float32),
            pltpu.VMEM((CHUNK, H), jnp.float32),
            pltpu.SemaphoreType.DMA,
            pltpu.SemaphoreType.DMA,
            pltpu.SemaphoreType.DMA,
            pltpu.SemaphoreType.DMA,
            pltpu.SemaphoreType.DMA,
            pltpu.SemaphoreType.DMA,
        ],
    )
    def gather_rows(qp_hbm, table_hbm, out_hbm,
                    cp_v, idx0_v, idx1_v, idx2_v,
                    rows0_v, rows1_v, rows2_v,
                    sg0, sg1, sg2, sw0, sw1, sw2):
        wid = lax.axis_index("s") * NC + lax.axis_index("c")
        base = wid * rows_per_w
        p_lo = lax.shift_right_logical(base, 12)
        pltpu.sync_copy(qp_hbm.at[p_lo], cp_v.at[pl.ds(0, B)])
        pltpu.sync_copy(qp_hbm.at[p_lo + 1], cp_v.at[pl.ds(B, B)])
        lanes = lax.iota(jnp.int32, L)

        idx_v = (idx0_v, idx1_v, idx2_v)
        rows_v = (rows0_v, rows1_v, rows2_v)
        sg = (sg0, sg1, sg2)
        sw = (sw0, sw1, sw2)

        def compute_idx(c, dst):
            start = base + c * CHUNK
            p = lax.shift_right_logical(start, 12)
            b0 = jnp.bitwise_and(start, B - 1)
            off = lax.shift_left(p - p_lo, 12) + b0
            for v in range(CHUNK // L):
                bv = b0 + v * L + lanes
                f = bv * _KK + p
                q = lax.shift_right_logical(f, 12)
                i = lax.shift_right_logical(q * 13108, 16)
                j = q - i * _K
                gxy = cp_v[pl.ds(off + v * L, L)]
                gx = jnp.bitwise_and(gxy, 0xFFFF)
                gy = lax.shift_right_logical(gxy, 16)
                rx = jnp.bitwise_and(gx + (i - _W), N - 1)
                ry = jnp.bitwise_and(gy + (j - _W), M - 1)
                dst[pl.ds(v * L, L)] = lax.shift_left(rx, 9) + ry

        def fire_gather(p):
            pltpu.async_copy(table_hbm.at[idx_v[p]], rows_v[p], sg[p])

        def wait_gather(p):
            pltpu.make_async_copy(table_hbm.at[idx_v[p]], rows_v[p],
                                  sg[p]).wait()

        def fire_wb(c, p):
            pltpu.async_copy(rows_v[p],
                             out_hbm.at[pl.ds(base + c * CHUNK, CHUNK)], sw[p])

        def wait_wb(c, p):
            pltpu.make_async_copy(rows_v[p],
                                  out_hbm.at[pl.ds(base + c * CHUNK, CHUNK)],
                                  sw[p]).wait()

        def step(c, p):
            wait_wb(c - 1, (p + 2) % 3)
            fire_gather((p + 2) % 3)
            wait_gather(p)
            fire_wb(c, p)
            compute_idx(c + 3, idx_v[p])

        compute_idx(0, idx_v[0])
        fire_gather(0)
        compute_idx(1, idx_v[1])
        fire_gather(1)
        compute_idx(2, idx_v[2])
        fire_gather(2)
        wait_gather(0)
        fire_wb(0, 0)
        compute_idx(3, idx_v[0])
        for c, p in ((1, 1), (2, 2)):
            step(c, p)

        def body(k, carry):
            c0 = k * 3
            step(c0, 0)
            step(c0 + 1, 1)
            step(c0 + 2, 2)
            return carry

        lax.fori_loop(1, 7, body, 0)
        wait_wb(20, 2)
        fire_gather(2)
        wait_gather(0)
        fire_wb(21, 0)
        compute_idx(24, idx_v[0])
        wait_wb(21, 0)
        fire_gather(0)
        wait_gather(1)
        fire_wb(22, 1)
        wait_wb(22, 1)
        wait_gather(2)
        fire_wb(23, 2)
        wait_wb(23, 2)
        wait_gather(0)
        fire_wb(24, 0)
        wait_wb(24, 0)

    out = gather_rows(qp, table)
    return jnp.transpose(out.reshape(_KK, B, H), (1, 0, 2))


# speedup vs baseline: 1.0689x; 1.0060x over previous
import functools

import jax
import jax.numpy as jnp
from jax import lax
from jax.experimental import pallas as pl
from jax.experimental.pallas import tpu as pltpu
from jax.experimental.pallas import tpu_sc as plsc

_W = 2
_K = 5
_KK = _K * _K


def kernel(grid_input, w, memory):
    B = grid_input.shape[0]
    N, M, H = memory.shape
    assert B == 4096 and N == 512 and M == 512 and H == 256
    table = memory.reshape(N * M, H)
    grid_adj = grid_input + (w - _W)
    qpk = grid_adj[:, 0] | (grid_adj[:, 1] << 16)
    qp = jnp.tile(qpk, _KK).reshape(B, _KK).T

    R = B * _KK
    NC, NS, L = 2, 16, 16
    NW = NC * NS
    rows_per_w = R // NW
    CHUNK = 128
    n_chunks = rows_per_w // CHUNK

    mesh = plsc.VectorSubcoreMesh(core_axis_name="c", subcore_axis_name="s")

    @functools.partial(
        pl.kernel,
        mesh=mesh,
        compiler_params=pltpu.CompilerParams(use_tc_tiling_on_sc=True),
        out_type=jax.ShapeDtypeStruct((R, H), jnp.float32),
        scratch_types=[
            pltpu.VMEM((2 * B,), jnp.int32),
            pltpu.VMEM((CHUNK,), jnp.int32),
            pltpu.VMEM((CHUNK,), jnp.int32),
            pltpu.VMEM((CHUNK,), jnp.int32),
            pltpu.VMEM((CHUNK, H), jnp.float32),
            pltpu.VMEM((CHUNK, H), jnp.float32),
            pltpu.VMEM((CHUNK, H), jnp.float32),
            pltpu.SemaphoreType.DMA,
            pltpu.SemaphoreType.DMA,
            pltpu.SemaphoreType.DMA,
            pltpu.SemaphoreType.DMA,
            pltpu.SemaphoreType.DMA,
            pltpu.SemaphoreType.DMA,
        ],
    )
    def gather_rows(qp_hbm, table_hbm, out_hbm,
                    cp_v, idx0_v, idx1_v, idx2_v,
                    rows0_v, rows1_v, rows2_v,
                    sg0, sg1, sg2, sw0, sw1, sw2):
        wid = lax.axis_index("s") * NC + lax.axis_index("c")
        base = wid * rows_per_w
        p_lo = lax.shift_right_logical(base, 12)
        pltpu.sync_copy(qp_hbm.at[p_lo], cp_v.at[pl.ds(0, B)])
        pltpu.sync_copy(qp_hbm.at[jnp.minimum(p_lo + 1, _KK - 1)],
                        cp_v.at[pl.ds(B, B)])
        lanes = lax.iota(jnp.int32, L)

        idx_v = (idx0_v, idx1_v, idx2_v)
        rows_v = (rows0_v, rows1_v, rows2_v)
        sg = (sg0, sg1, sg2)
        sw = (sw0, sw1, sw2)

        def compute_idx(c, dst):
            start = base + c * CHUNK
            p = lax.shift_right_logical(start, 12)
            b0 = jnp.bitwise_and(start, B - 1)
            off = lax.shift_left(p - p_lo, 12) + b0
            for v in range(CHUNK // L):
                bv = b0 + v * L + lanes
                f = bv * _KK + p
                q = lax.shift_right_logical(f, 12)
                i = lax.shift_right_logical(q * 13108, 16)
                j = q - i * _K
                gxy = cp_v[pl.ds(off + v * L, L)]
                gx = jnp.bitwise_and(gxy, 0xFFFF)
                gy = lax.shift_right_logical(gxy, 16)
                rx = jnp.bitwise_and(gx + (i - _W), N - 1)
                ry = jnp.bitwise_and(gy + (j - _W), M - 1)
                dst[pl.ds(v * L, L)] = lax.shift_left(rx, 9) + ry

        def fire_gather(p):
            pltpu.async_copy(table_hbm.at[idx_v[p]], rows_v[p], sg[p])

        def wait_gather(p):
            pltpu.make_async_copy(table_hbm.at[idx_v[p]], rows_v[p],
                                  sg[p]).wait()

        def fire_wb(c, p):
            pltpu.async_copy(rows_v[p],
                             out_hbm.at[pl.ds(base + c * CHUNK, CHUNK)], sw[p])

        def wait_wb(c, p):
            pltpu.make_async_copy(rows_v[p],
                                  out_hbm.at[pl.ds(base + c * CHUNK, CHUNK)],
                                  sw[p]).wait()

        def step(c, p):
            wait_wb(c - 1, (p + 2) % 3)
            fire_gather((p + 2) % 3)
            wait_gather(p)
            fire_wb(c, p)
            compute_idx(c + 3, idx_v[p])

        compute_idx(0, idx_v[0])
        fire_gather(0)
        compute_idx(1, idx_v[1])
        fire_gather(1)
        compute_idx(2, idx_v[2])
        fire_gather(2)
        wait_gather(0)
        fire_wb(0, 0)
        compute_idx(3, idx_v[0])
        for c, p in ((1, 1), (2, 2)):
            step(c, p)

        def body(k, carry):
            c0 = k * 3
            step(c0, 0)
            step(c0 + 1, 1)
            step(c0 + 2, 2)
            return carry

        lax.fori_loop(1, 7, body, 0)
        wait_wb(20, 2)
        fire_gather(2)
        wait_gather(0)
        fire_wb(21, 0)
        compute_idx(24, idx_v[0])
        wait_wb(21, 0)
        fire_gather(0)
        wait_gather(1)
        fire_wb(22, 1)
        wait_wb(22, 1)
        wait_gather(2)
        fire_wb(23, 2)
        wait_wb(23, 2)
        wait_gather(0)
        fire_wb(24, 0)
        wait_wb(24, 0)

    out = gather_rows(qp, table)
    return jnp.transpose(out.reshape(_KK, B, H), (1, 0, 2))


# speedup vs baseline: 1.0804x; 1.0107x over previous
import functools

import jax
import jax.numpy as jnp
from jax import lax
from jax.experimental import pallas as pl
from jax.experimental.pallas import tpu as pltpu
from jax.experimental.pallas import tpu_sc as plsc

_W = 2
_K = 5
_KK = _K * _K


def kernel(grid_input, w, memory):
    B = grid_input.shape[0]
    N, M, H = memory.shape
    assert B == 4096 and N == 512 and M == 512 and H == 256
    table = memory.reshape(N * M, H)
    grid_adj = grid_input + (w - _W)
    qpk = grid_adj[:, 0] | (grid_adj[:, 1] << 16)
    qp = jnp.tile(qpk, _KK).reshape(B, _KK).T

    R = B * _KK
    NC, NS, L = 2, 16, 16
    NW = NC * NS
    rows_per_w = R // NW
    CHUNK = 128
    n_chunks = rows_per_w // CHUNK

    mesh = plsc.VectorSubcoreMesh(core_axis_name="c", subcore_axis_name="s")

    @functools.partial(
        pl.kernel,
        mesh=mesh,
        compiler_params=pltpu.CompilerParams(use_tc_tiling_on_sc=True),
        out_type=jax.ShapeDtypeStruct((R, H), jnp.float32),
        scratch_types=[
            pltpu.VMEM((2 * B,), jnp.int32),
            pltpu.VMEM((CHUNK,), jnp.int32),
            pltpu.VMEM((CHUNK,), jnp.int32),
            pltpu.VMEM((CHUNK,), jnp.int32),
            pltpu.VMEM((CHUNK, H), jnp.float32),
            pltpu.VMEM((CHUNK, H), jnp.float32),
            pltpu.VMEM((CHUNK, H), jnp.float32),
            pltpu.SemaphoreType.DMA,
            pltpu.SemaphoreType.DMA,
            pltpu.SemaphoreType.DMA,
            pltpu.SemaphoreType.DMA,
            pltpu.SemaphoreType.DMA,
            pltpu.SemaphoreType.DMA,
        ],
    )
    def gather_rows(qp_hbm, table_hbm, out_hbm,
                    cp_v, idx0_v, idx1_v, idx2_v,
                    rows0_v, rows1_v, rows2_v,
                    sg0, sg1, sg2, sw0, sw1, sw2):
        wid = lax.axis_index("s") * NC + lax.axis_index("c")
        base = wid * rows_per_w
        p_lo = lax.shift_right_logical(base, 12)
        p_hi = jnp.minimum(p_lo + 1, _KK - 1)
        cpy0 = pltpu.async_copy(qp_hbm.at[p_lo], cp_v.at[pl.ds(0, B)], sg0)
        cpy1 = pltpu.async_copy(qp_hbm.at[p_hi], cp_v.at[pl.ds(B, B)], sg1)
        cpy0.wait()
        cpy1.wait()
        lanes = lax.iota(jnp.int32, L)

        idx_v = (idx0_v, idx1_v, idx2_v)
        rows_v = (rows0_v, rows1_v, rows2_v)
        sg = (sg0, sg1, sg2)
        sw = (sw0, sw1, sw2)

        def compute_idx(c, dst):
            start = base + c * CHUNK
            p = lax.shift_right_logical(start, 12)
            b0 = jnp.bitwise_and(start, B - 1)
            off = lax.shift_left(p - p_lo, 12) + b0
            for v in range(CHUNK // L):
                bv = b0 + v * L + lanes
                f = bv * _KK + p
                q = lax.shift_right_logical(f, 12)
                i = lax.shift_right_logical(q * 13108, 16)
                j = q - i * _K
                gxy = cp_v[pl.ds(off + v * L, L)]
                gx = jnp.bitwise_and(gxy, 0xFFFF)
                gy = lax.shift_right_logical(gxy, 16)
                rx = jnp.bitwise_and(gx + (i - _W), N - 1)
                ry = jnp.bitwise_and(gy + (j - _W), M - 1)
                dst[pl.ds(v * L, L)] = lax.shift_left(rx, 9) + ry

        def fire_gather(p):
            pltpu.async_copy(table_hbm.at[idx_v[p]], rows_v[p], sg[p])

        def wait_gather(p):
            pltpu.make_async_copy(table_hbm.at[idx_v[p]], rows_v[p],
                                  sg[p]).wait()

        def fire_wb(c, p):
            pltpu.async_copy(rows_v[p],
                             out_hbm.at[pl.ds(base + c * CHUNK, CHUNK)], sw[p])

        def wait_wb(c, p):
            pltpu.make_async_copy(rows_v[p],
                                  out_hbm.at[pl.ds(base + c * CHUNK, CHUNK)],
                                  sw[p]).wait()

        def step(c, p):
            wait_wb(c - 1, (p + 2) % 3)
            fire_gather((p + 2) % 3)
            wait_gather(p)
            fire_wb(c, p)
            compute_idx(c + 3, idx_v[p])

        compute_idx(0, idx_v[0])
        fire_gather(0)
        compute_idx(1, idx_v[1])
        fire_gather(1)
        compute_idx(2, idx_v[2])
        fire_gather(2)
        wait_gather(0)
        fire_wb(0, 0)
        compute_idx(3, idx_v[0])
        for c, p in ((1, 1), (2, 2)):
            step(c, p)

        def body(k, carry):
            c0 = k * 3
            step(c0, 0)
            step(c0 + 1, 1)
            step(c0 + 2, 2)
            return carry

        lax.fori_loop(1, 7, body, 0)
        wait_wb(20, 2)
        fire_gather(2)
        wait_gather(0)
        fire_wb(21, 0)
        compute_idx(24, idx_v[0])
        wait_wb(21, 0)
        fire_gather(0)
        wait_gather(1)
        fire_wb(22, 1)
        wait_wb(22, 1)
        wait_gather(2)
        fire_wb(23, 2)
        wait_wb(23, 2)
        wait_gather(0)
        fire_wb(24, 0)
        wait_wb(24, 0)

    out = gather_rows(qp, table)
    return jnp.transpose(out.reshape(_KK, B, H), (1, 0, 2))
